# baseline (speedup 1.0000x reference)
import functools

import jax
import jax.numpy as jnp
from jax import lax
from jax.experimental import pallas as pl
from jax.experimental.pallas import tpu as pltpu
from jax.experimental.pallas import tpu_sc as plsc

N = 2048
C = 128
K = 20

RB = 512
_EPS = 1e-8

_NC = 2
_NS = 16
_NW = _NC * _NS
_NPW = N // _NW
_CH = 4
_EC = _CH * K
_NCHUNK = _NPW // _CH
_LANES = 16


def _dot(a, b, dims=(((1,), (0,)), ((), ()))):
    return lax.dot_general(a.astype(jnp.bfloat16), b.astype(jnp.bfloat16),
                           dims, preferred_element_type=jnp.float32)


def _stage1_body(feat_blk_ref, feat_full_ref, w1t_ref, w1b_ref, b1_ref,
                 idx_ref, a_ref, g_ref):
    feat_full = feat_full_ref[...]
    norm_full = jnp.sqrt(jnp.sum(feat_full * feat_full, axis=1, keepdims=True))
    fn_full = feat_full / jnp.maximum(norm_full, _EPS)

    feat_blk = feat_blk_ref[...]
    norm_blk = jnp.sqrt(jnp.sum(feat_blk * feat_blk, axis=1, keepdims=True))
    fn_blk = feat_blk / jnp.maximum(norm_blk, _EPS)

    sim = _dot(fn_blk, fn_full, (((1,), (1,)), ((), ()))) + 1.0

    icols = lax.broadcasted_iota(jnp.int32, (N, 1), 0)
    colmat = jnp.concatenate(
        [(icols >> 8).astype(jnp.float32), (icols & 255).astype(jnp.float32)],
        axis=1)
    neg_inf = jnp.float32(-jnp.inf)
    picked = []
    for _ in range(K):
        m = jnp.max(sim, axis=1, keepdims=True)
        eq = sim == m
        fmask = jnp.where(eq, 1.0, 0.0)
        sim = jnp.where(eq, neg_inf, sim)
        hv_lv = _dot(fmask, colmat)
        picked.append(hv_lv[:, 0:1] * 256.0 + hv_lv[:, 1:2])
    amin = jnp.minimum(jnp.concatenate(picked, axis=1), jnp.float32(N - 1))
    idx_ref[...] = amin.astype(jnp.int32)

    a_ref[...] = _dot(feat_blk, w1t_ref[...]) + b1_ref[...]
    g_ref[...] = _dot(feat_blk, w1b_ref[...])


_stage1 = pl.pallas_call(
    _stage1_body,
    grid=(N // RB,),
    in_specs=[
        pl.BlockSpec((RB, C), lambda i: (i, 0)),
        pl.BlockSpec((N, C), lambda i: (0, 0)),
        pl.BlockSpec((C, C), lambda i: (0, 0)),
        pl.BlockSpec((C, C), lambda i: (0, 0)),
        pl.BlockSpec((1, C), lambda i: (0, 0)),
    ],
    out_specs=[
        pl.BlockSpec((RB, K), lambda i: (i, 0)),
        pl.BlockSpec((RB, C), lambda i: (i, 0)),
        pl.BlockSpec((RB, C), lambda i: (i, 0)),
    ],
    out_shape=[
        jax.ShapeDtypeStruct((N, K), jnp.int32),
        jax.ShapeDtypeStruct((N, C), jnp.float32),
        jax.ShapeDtypeStruct((N, C), jnp.float32),
    ],
)


_NBUF = 4


def _sc_gather_body(idx_hbm, a_hbm, g_hbm, out_hbm,
                    idx_v, a_v, rows0, rows1, rows2, rows3, o_v,
                    sem0, sem1, sem2, sem3):
    wid = lax.axis_index("s") * _NC + lax.axis_index("c")
    base = wid * _NPW
    pltpu.sync_copy(idx_hbm.at[wid], idx_v)
    pltpu.sync_copy(a_hbm.at[pl.ds(base, _NPW)], a_v)

    bufs = (rows0, rows1, rows2, rows3)
    sems = (sem0, sem1, sem2, sem3)

    def compute(c, buf):
        for n in range(_CH):
            node = c * _CH + n
            for cb in range(C // _LANES):
                sl = pl.ds(cb * _LANES, _LANES)
                a_vec = a_v[node, sl]
                acc = jnp.zeros((_LANES,), jnp.float32)
                for k in range(K):
                    acc = acc + jnp.maximum(a_vec + buf[n * K + k, sl], 0.0)
                o_v[node, sl] = acc

    for p in range(_NBUF - 1):
        pltpu.async_copy(g_hbm.at[idx_v.at[p]], bufs[p], sems[p])

    def quad_body(i, carry):
        c0 = _NBUF * i
        for b in range(_NBUF):
            c = c0 + b
            nxt = c + (_NBUF - 1)
            nslot = (b + _NBUF - 1) % _NBUF

            @pl.when(nxt < _NCHUNK)
            def _(nxt=nxt, nslot=nslot):
                pltpu.async_copy(g_hbm.at[idx_v.at[nxt]], bufs[nslot],
                                 sems[nslot])

            pltpu.make_async_copy(g_hbm.at[idx_v.at[c]], bufs[b],
                                  sems[b]).wait()
            compute(c, bufs[b])
        return carry

    lax.fori_loop(0, _NCHUNK // _NBUF, quad_body, 0)
    pltpu.sync_copy(o_v, out_hbm.at[pl.ds(base, _NPW)])


@functools.lru_cache(maxsize=1)
def _sc_gather():
    return pl.kernel(
        _sc_gather_body,
        mesh=plsc.VectorSubcoreMesh(core_axis_name="c", subcore_axis_name="s"),
        out_type=jax.ShapeDtypeStruct((N, C), jnp.float32),
        scratch_types=[
            pltpu.VMEM((_NCHUNK, _EC), jnp.int32),
            pltpu.VMEM((_NPW, C), jnp.float32),
            pltpu.VMEM((_EC, C), jnp.float32),
            pltpu.VMEM((_EC, C), jnp.float32),
            pltpu.VMEM((_EC, C), jnp.float32),
            pltpu.VMEM((_EC, C), jnp.float32),
            pltpu.VMEM((_NPW, C), jnp.float32),
            pltpu.SemaphoreType.DMA,
            pltpu.SemaphoreType.DMA,
            pltpu.SemaphoreType.DMA,
            pltpu.SemaphoreType.DMA,
        ],
    )


def _stage3_body(feat_ref, hpre_ref, w2_ref, b2_ref, u1t_ref, u1b_ref,
                 ub1_ref, u2_ref, ub2_ref, out_ref):
    feat = feat_ref[...]
    hagg = _dot(hpre_ref[...], w2_ref[...]) + jnp.float32(K) * b2_ref[...]
    hh = jnp.maximum(
        _dot(feat, u1t_ref[...]) + _dot(hagg, u1b_ref[...]) + ub1_ref[...],
        0.0)
    out_ref[...] = feat + _dot(hh, u2_ref[...]) + ub2_ref[...]


_stage3 = pl.pallas_call(
    _stage3_body,
    out_shape=jax.ShapeDtypeStruct((N, C), jnp.float32),
)


def kernel(feature, msg_w1, msg_b1, msg_w2, msg_b2,
           upd_w1, upd_b1, upd_w2, upd_b2):
    idx, a, g = _stage1(feature, feature, msg_w1[:C], msg_w1[C:],
                        msg_b1.reshape(1, C))
    idx3 = idx.reshape(_NW, _NCHUNK, _EC)
    hpre = _sc_gather()(idx3, a, g)
    out = _stage3(feature, hpre, msg_w2, msg_b2.reshape(1, C),
                  upd_w1[:C], upd_w1[C:], upd_b1.reshape(1, C),
                  upd_w2, upd_b2.reshape(1, C))
    return out


# speedup vs baseline: 1.0477x; 1.0477x over previous
import functools

import jax
import jax.numpy as jnp
from jax import lax
from jax.experimental import pallas as pl
from jax.experimental.pallas import tpu as pltpu
from jax.experimental.pallas import tpu_sc as plsc

N = 2048
C = 128
K = 20

RB = 512
_EPS = 1e-8

_NC = 2
_NS = 16
_NW = _NC * _NS
_NPW = N // _NW
_CH = 4
_EC = _CH * K
_NCHUNK = _NPW // _CH
_LANES = 16


def _dot(a, b, dims=(((1,), (0,)), ((), ()))):
    return lax.dot_general(a.astype(jnp.bfloat16), b.astype(jnp.bfloat16),
                           dims, preferred_element_type=jnp.float32)


def _stage1_body(feat_blk_ref, feat_full_ref, w1t_ref, w1b_ref, b1_ref,
                 idx_ref, a_ref, g_ref):
    feat_full = feat_full_ref[...]
    norm_full = jnp.sqrt(jnp.sum(feat_full * feat_full, axis=1, keepdims=True))
    fn_full = feat_full / jnp.maximum(norm_full, _EPS)

    feat_blk = feat_blk_ref[...]
    norm_blk = jnp.sqrt(jnp.sum(feat_blk * feat_blk, axis=1, keepdims=True))
    fn_blk = feat_blk / jnp.maximum(norm_blk, _EPS)

    sim = _dot(fn_blk, fn_full, (((1,), (1,)), ((), ()))) + 1.0

    cols = lax.broadcasted_iota(jnp.int32, (RB, N), 1).astype(jnp.float32)
    neg_inf = jnp.float32(-jnp.inf)
    big = jnp.float32(N)
    picked = []
    for _ in range(K):
        m = jnp.max(sim, axis=1, keepdims=True)
        amin = jnp.min(jnp.where(sim == m, cols, big), axis=1, keepdims=True)
        picked.append(amin)
        sim = jnp.where(cols == amin, neg_inf, sim)
    idx_ref[...] = jnp.concatenate(picked, axis=1).astype(jnp.int32)

    a_ref[...] = _dot(feat_blk, w1t_ref[...]) + b1_ref[...]
    g_ref[...] = _dot(feat_blk, w1b_ref[...])


_stage1 = pl.pallas_call(
    _stage1_body,
    grid=(N // RB,),
    in_specs=[
        pl.BlockSpec((RB, C), lambda i: (i, 0)),
        pl.BlockSpec((N, C), lambda i: (0, 0)),
        pl.BlockSpec((C, C), lambda i: (0, 0)),
        pl.BlockSpec((C, C), lambda i: (0, 0)),
        pl.BlockSpec((1, C), lambda i: (0, 0)),
    ],
    out_specs=[
        pl.BlockSpec((RB, K), lambda i: (i, 0)),
        pl.BlockSpec((RB, C), lambda i: (i, 0)),
        pl.BlockSpec((RB, C), lambda i: (i, 0)),
    ],
    out_shape=[
        jax.ShapeDtypeStruct((N, K), jnp.int32),
        jax.ShapeDtypeStruct((N, C), jnp.float32),
        jax.ShapeDtypeStruct((N, C), jnp.float32),
    ],
)


_NBUF = 4


def _sc_gather_body(idx_hbm, a_hbm, g_hbm, out_hbm,
                    idx_v, a_v, rows0, rows1, rows2, rows3, o_v,
                    sem0, sem1, sem2, sem3):
    wid = lax.axis_index("s") * _NC + lax.axis_index("c")
    base = wid * _NPW
    pltpu.sync_copy(idx_hbm.at[wid], idx_v)
    pltpu.sync_copy(a_hbm.at[pl.ds(base, _NPW)], a_v)

    bufs = (rows0, rows1, rows2, rows3)
    sems = (sem0, sem1, sem2, sem3)

    def compute(c, buf):
        for n in range(_CH):
            node = c * _CH + n
            for cb in range(C // _LANES):
                sl = pl.ds(cb * _LANES, _LANES)
                a_vec = a_v[node, sl]
                acc = jnp.zeros((_LANES,), jnp.float32)
                for k in range(K):
                    acc = acc + jnp.maximum(a_vec + buf[n * K + k, sl], 0.0)
                o_v[node, sl] = acc

    for p in range(_NBUF - 1):
        pltpu.async_copy(g_hbm.at[idx_v.at[p]], bufs[p], sems[p])

    def quad_body(i, carry):
        c0 = _NBUF * i
        for b in range(_NBUF):
            c = c0 + b
            nxt = c + (_NBUF - 1)
            nslot = (b + _NBUF - 1) % _NBUF

            @pl.when(nxt < _NCHUNK)
            def _(nxt=nxt, nslot=nslot):
                pltpu.async_copy(g_hbm.at[idx_v.at[nxt]], bufs[nslot],
                                 sems[nslot])

            pltpu.make_async_copy(g_hbm.at[idx_v.at[c]], bufs[b],
                                  sems[b]).wait()
            compute(c, bufs[b])
        return carry

    lax.fori_loop(0, _NCHUNK // _NBUF, quad_body, 0)
    pltpu.sync_copy(o_v, out_hbm.at[pl.ds(base, _NPW)])


@functools.lru_cache(maxsize=1)
def _sc_gather():
    return pl.kernel(
        _sc_gather_body,
        mesh=plsc.VectorSubcoreMesh(core_axis_name="c", subcore_axis_name="s"),
        out_type=jax.ShapeDtypeStruct((N, C), jnp.float32),
        scratch_types=[
            pltpu.VMEM((_NCHUNK, _EC), jnp.int32),
            pltpu.VMEM((_NPW, C), jnp.float32),
            pltpu.VMEM((_EC, C), jnp.float32),
            pltpu.VMEM((_EC, C), jnp.float32),
            pltpu.VMEM((_EC, C), jnp.float32),
            pltpu.VMEM((_EC, C), jnp.float32),
            pltpu.VMEM((_NPW, C), jnp.float32),
            pltpu.SemaphoreType.DMA,
            pltpu.SemaphoreType.DMA,
            pltpu.SemaphoreType.DMA,
            pltpu.SemaphoreType.DMA,
        ],
    )


def _stage3_body(feat_ref, hpre_ref, w2_ref, b2_ref, u1t_ref, u1b_ref,
                 ub1_ref, u2_ref, ub2_ref, out_ref):
    feat = feat_ref[...]
    hagg = _dot(hpre_ref[...], w2_ref[...]) + jnp.float32(K) * b2_ref[...]
    hh = jnp.maximum(
        _dot(feat, u1t_ref[...]) + _dot(hagg, u1b_ref[...]) + ub1_ref[...],
        0.0)
    out_ref[...] = feat + _dot(hh, u2_ref[...]) + ub2_ref[...]


_stage3 = pl.pallas_call(
    _stage3_body,
    out_shape=jax.ShapeDtypeStruct((N, C), jnp.float32),
)


def kernel(feature, msg_w1, msg_b1, msg_w2, msg_b2,
           upd_w1, upd_b1, upd_w2, upd_b2):
    idx, a, g = _stage1(feature, feature, msg_w1[:C], msg_w1[C:],
                        msg_b1.reshape(1, C))
    idx3 = idx.reshape(_NW, _NCHUNK, _EC)
    hpre = _sc_gather()(idx3, a, g)
    out = _stage3(feature, hpre, msg_w2, msg_b2.reshape(1, C),
                  upd_w1[:C], upd_w1[C:], upd_b1.reshape(1, C),
                  upd_w2, upd_b2.reshape(1, C))
    return out


# speedup vs baseline: 1.1300x; 1.0786x over previous
import functools

import jax
import jax.numpy as jnp
from jax import lax
from jax.experimental import pallas as pl
from jax.experimental.pallas import tpu as pltpu
from jax.experimental.pallas import tpu_sc as plsc

N = 2048
C = 128
K = 20

RB = 512
_EPS = 1e-8

_NC = 2
_NS = 16
_NW = _NC * _NS
_NPW = N // _NW
_CH = 4
_EC = _CH * K
_NCHUNK = _NPW // _CH
_LANES = 16


def _dot(a, b, dims=(((1,), (0,)), ((), ()))):
    return lax.dot_general(a.astype(jnp.bfloat16), b.astype(jnp.bfloat16),
                           dims, preferred_element_type=jnp.float32)


def _stage1_body(feat_blk_ref, feat_full_ref, w1t_ref, w1b_ref, b1_ref,
                 idx_ref, a_ref, g_ref):
    feat_full = feat_full_ref[...]
    norm_full = jnp.sqrt(jnp.sum(feat_full * feat_full, axis=1, keepdims=True))
    fn_full = feat_full / jnp.maximum(norm_full, _EPS)

    feat_blk = feat_blk_ref[...]
    norm_blk = jnp.sqrt(jnp.sum(feat_blk * feat_blk, axis=1, keepdims=True))
    fn_blk = feat_blk / jnp.maximum(norm_blk, _EPS)

    sim = _dot(fn_blk, fn_full, (((1,), (1,)), ((), ()))) + 1.0

    cols = lax.broadcasted_iota(jnp.int32, (RB, N), 1).astype(jnp.float32)
    neg_inf = jnp.float32(-jnp.inf)
    big = jnp.float32(N)
    picked = []
    for _ in range(K):
        m = jnp.max(sim, axis=1, keepdims=True)
        amin = jnp.min(jnp.where(sim == m, cols, big), axis=1, keepdims=True)
        picked.append(amin)
        sim = jnp.where(cols == amin, neg_inf, sim)
    idx_ref[...] = jnp.concatenate(picked, axis=1).astype(jnp.int32)

    a_ref[...] = _dot(feat_blk, w1t_ref[...]) + b1_ref[...]
    g_ref[...] = _dot(feat_blk, w1b_ref[...])


_stage1 = pl.pallas_call(
    _stage1_body,
    grid=(N // RB,),
    in_specs=[
        pl.BlockSpec((RB, C), lambda i: (i, 0)),
        pl.BlockSpec((N, C), lambda i: (0, 0)),
        pl.BlockSpec((C, C), lambda i: (0, 0)),
        pl.BlockSpec((C, C), lambda i: (0, 0)),
        pl.BlockSpec((1, C), lambda i: (0, 0)),
    ],
    out_specs=[
        pl.BlockSpec((RB, K), lambda i: (i, 0)),
        pl.BlockSpec((RB, C), lambda i: (i, 0)),
        pl.BlockSpec((RB, C), lambda i: (i, 0)),
    ],
    out_shape=[
        jax.ShapeDtypeStruct((N, K), jnp.int32),
        jax.ShapeDtypeStruct((N, C), jnp.float32),
        jax.ShapeDtypeStruct((N, C), jnp.float32),
    ],
)


def _sc_gather_body(idx_hbm, a_hbm, g_hbm, out_hbm,
                    idx_v, a_v, rows0, rows1, o_v, sem0, sem1):
    wid = lax.axis_index("s") * _NC + lax.axis_index("c")
    base = wid * _NPW
    pltpu.sync_copy(idx_hbm.at[wid], idx_v)
    pltpu.sync_copy(a_hbm.at[pl.ds(base, _NPW)], a_v)

    def compute(c, buf):
        for n in range(_CH):
            node = c * _CH + n
            for cb in range(C // _LANES):
                sl = pl.ds(cb * _LANES, _LANES)
                a_vec = a_v[node, sl]
                acc = jnp.zeros((_LANES,), jnp.float32)
                for k in range(K):
                    acc = acc + jnp.maximum(a_vec + buf[n * K + k, sl], 0.0)
                o_v[node, sl] = acc

    pltpu.async_copy(g_hbm.at[idx_v.at[0]], rows0, sem0)

    def pair_body(i, carry):
        c0 = 2 * i
        c1 = c0 + 1
        pltpu.async_copy(g_hbm.at[idx_v.at[c1]], rows1, sem1)
        pltpu.make_async_copy(g_hbm.at[idx_v.at[c0]], rows0, sem0).wait()
        compute(c0, rows0)

        @pl.when(c1 + 1 < _NCHUNK)
        def _():
            pltpu.async_copy(g_hbm.at[idx_v.at[c1 + 1]], rows0, sem0)

        pltpu.make_async_copy(g_hbm.at[idx_v.at[c1]], rows1, sem1).wait()
        compute(c1, rows1)
        return carry

    lax.fori_loop(0, _NCHUNK // 2, pair_body, 0)
    pltpu.sync_copy(o_v, out_hbm.at[pl.ds(base, _NPW)])


@functools.lru_cache(maxsize=1)
def _sc_gather():
    return pl.kernel(
        _sc_gather_body,
        mesh=plsc.VectorSubcoreMesh(core_axis_name="c", subcore_axis_name="s"),
        out_type=jax.ShapeDtypeStruct((N, C), jnp.float32),
        scratch_types=[
            pltpu.VMEM((_NCHUNK, _EC), jnp.int32),
            pltpu.VMEM((_NPW, C), jnp.float32),
            pltpu.VMEM((_EC, C), jnp.float32),
            pltpu.VMEM((_EC, C), jnp.float32),
            pltpu.VMEM((_NPW, C), jnp.float32),
            pltpu.SemaphoreType.DMA,
            pltpu.SemaphoreType.DMA,
        ],
    )


def _stage3_body(feat_ref, hpre_ref, w2_ref, b2_ref, u1t_ref, u1b_ref,
                 ub1_ref, u2_ref, ub2_ref, out_ref):
    feat = feat_ref[...]
    hagg = _dot(hpre_ref[...], w2_ref[...]) + jnp.float32(K) * b2_ref[...]
    hh = jnp.maximum(
        _dot(feat, u1t_ref[...]) + _dot(hagg, u1b_ref[...]) + ub1_ref[...],
        0.0)
    out_ref[...] = feat + _dot(hh, u2_ref[...]) + ub2_ref[...]


_stage3 = pl.pallas_call(
    _stage3_body,
    out_shape=jax.ShapeDtypeStruct((N, C), jnp.float32),
)


def kernel(feature, msg_w1, msg_b1, msg_w2, msg_b2,
           upd_w1, upd_b1, upd_w2, upd_b2):
    idx, a, g = _stage1(feature, feature, msg_w1[:C], msg_w1[C:],
                        msg_b1.reshape(1, C))
    idx3 = idx.reshape(_NW, _NCHUNK, _EC)
    hpre = _sc_gather()(idx3, a, g)
    out = _stage3(feature, hpre, msg_w2, msg_b2.reshape(1, C),
                  upd_w1[:C], upd_w1[C:], upd_b1.reshape(1, C),
                  upd_w2, upd_b2.reshape(1, C))
    return out


# speedup vs baseline: 1.1510x; 1.0186x over previous
import functools

import jax
import jax.numpy as jnp
from jax import lax
from jax.experimental import pallas as pl
from jax.experimental.pallas import tpu as pltpu
from jax.experimental.pallas import tpu_sc as plsc

N = 2048
C = 128
K = 20

RB = 512
_NB = N // RB
_EPS = 1e-8

_NC = 2
_NS = 16
_NW = _NC * _NS
_NPW = RB // _NW
_CH = 4
_EC = _CH * K
_NCHUNK = _NPW // _CH
_LANES = 16


def _dot(a, b, dims=(((1,), (0,)), ((), ()))):
    return lax.dot_general(a.astype(jnp.bfloat16), b.astype(jnp.bfloat16),
                           dims, preferred_element_type=jnp.float32)


def _proj_body(feat_ref, w1t_ref, w1b_ref, b1_ref, a_ref, g_ref):
    feat = feat_ref[...]
    a_ref[...] = _dot(feat, w1t_ref[...]) + b1_ref[...]
    g_ref[...] = _dot(feat, w1b_ref[...])


_proj = pl.pallas_call(
    _proj_body,
    out_shape=[
        jax.ShapeDtypeStruct((N, C), jnp.float32),
        jax.ShapeDtypeStruct((N, C), jnp.float32),
    ],
)


def _topk_body(feat_blk_ref, feat_full_ref, idx_ref):
    feat_full = feat_full_ref[...]
    norm_full = jnp.sqrt(jnp.sum(feat_full * feat_full, axis=1, keepdims=True))
    fn_full = feat_full / jnp.maximum(norm_full, _EPS)

    feat_blk = feat_blk_ref[...]
    norm_blk = jnp.sqrt(jnp.sum(feat_blk * feat_blk, axis=1, keepdims=True))
    fn_blk = feat_blk / jnp.maximum(norm_blk, _EPS)

    sim = _dot(fn_blk, fn_full, (((1,), (1,)), ((), ()))) + 1.0

    cols = lax.broadcasted_iota(jnp.int32, (RB, N), 1).astype(jnp.float32)
    neg_inf = jnp.float32(-jnp.inf)
    big = jnp.float32(N)
    picked = []
    for _ in range(K):
        m = jnp.max(sim, axis=1, keepdims=True)
        amin = jnp.min(jnp.where(sim == m, cols, big), axis=1, keepdims=True)
        picked.append(amin)
        sim = jnp.where(cols == amin, neg_inf, sim)
    idx_ref[...] = jnp.concatenate(picked, axis=1).astype(jnp.int32)


_topk = pl.pallas_call(
    _topk_body,
    out_shape=jax.ShapeDtypeStruct((RB, K), jnp.int32),
)


def _sc_gather_body(idx_hbm, a_hbm, g_hbm, out_hbm,
                    idx_v, a_v, rows0, rows1, o_v, sem0, sem1):
    wid = lax.axis_index("s") * _NC + lax.axis_index("c")
    base = wid * _NPW
    pltpu.sync_copy(idx_hbm.at[wid], idx_v)
    pltpu.sync_copy(a_hbm.at[pl.ds(base, _NPW)], a_v)

    def compute(c, buf):
        for n in range(_CH):
            node = c * _CH + n
            for cb in range(C // _LANES):
                sl = pl.ds(cb * _LANES, _LANES)
                a_vec = a_v[node, sl]
                acc = jnp.zeros((_LANES,), jnp.float32)
                for k in range(K):
                    acc = acc + jnp.maximum(a_vec + buf[n * K + k, sl], 0.0)
                o_v[node, sl] = acc

    pltpu.async_copy(g_hbm.at[idx_v.at[0]], rows0, sem0)

    def pair_body(i, carry):
        c0 = 2 * i
        c1 = c0 + 1
        pltpu.async_copy(g_hbm.at[idx_v.at[c1]], rows1, sem1)
        pltpu.make_async_copy(g_hbm.at[idx_v.at[c0]], rows0, sem0).wait()
        compute(c0, rows0)

        @pl.when(c1 + 1 < _NCHUNK)
        def _():
            pltpu.async_copy(g_hbm.at[idx_v.at[c1 + 1]], rows0, sem0)

        pltpu.make_async_copy(g_hbm.at[idx_v.at[c1]], rows1, sem1).wait()
        compute(c1, rows1)
        return carry

    lax.fori_loop(0, _NCHUNK // 2, pair_body, 0)
    pltpu.sync_copy(o_v, out_hbm.at[pl.ds(base, _NPW)])


@functools.lru_cache(maxsize=1)
def _sc_gather():
    return pl.kernel(
        _sc_gather_body,
        mesh=plsc.VectorSubcoreMesh(core_axis_name="c", subcore_axis_name="s"),
        out_type=jax.ShapeDtypeStruct((RB, C), jnp.float32),
        scratch_types=[
            pltpu.VMEM((_NCHUNK, _EC), jnp.int32),
            pltpu.VMEM((_NPW, C), jnp.float32),
            pltpu.VMEM((_EC, C), jnp.float32),
            pltpu.VMEM((_EC, C), jnp.float32),
            pltpu.VMEM((_NPW, C), jnp.float32),
            pltpu.SemaphoreType.DMA,
            pltpu.SemaphoreType.DMA,
        ],
    )


def _stage3_body(feat_ref, hpre_ref, w2_ref, b2_ref, u1t_ref, u1b_ref,
                 ub1_ref, u2_ref, ub2_ref, out_ref):
    feat = feat_ref[...]
    hagg = _dot(hpre_ref[...], w2_ref[...]) + jnp.float32(K) * b2_ref[...]
    hh = jnp.maximum(
        _dot(feat, u1t_ref[...]) + _dot(hagg, u1b_ref[...]) + ub1_ref[...],
        0.0)
    out_ref[...] = feat + _dot(hh, u2_ref[...]) + ub2_ref[...]


_stage3 = pl.pallas_call(
    _stage3_body,
    out_shape=jax.ShapeDtypeStruct((N, C), jnp.float32),
)


def kernel(feature, msg_w1, msg_b1, msg_w2, msg_b2,
           upd_w1, upd_b1, upd_w2, upd_b2):
    a, g = _proj(feature, msg_w1[:C], msg_w1[C:], msg_b1.reshape(1, C))
    sc = _sc_gather()
    hpres = []
    for b in range(_NB):
        feat_b = lax.slice(feature, (b * RB, 0), ((b + 1) * RB, C))
        idx_b = _topk(feat_b, feature)
        a_b = lax.slice(a, (b * RB, 0), ((b + 1) * RB, C))
        hpres.append(sc(idx_b.reshape(_NW, _NCHUNK, _EC), a_b, g))
    hpre = jnp.concatenate(hpres, axis=0)
    out = _stage3(feature, hpre, msg_w2, msg_b2.reshape(1, C),
                  upd_w1[:C], upd_w1[C:], upd_b1.reshape(1, C),
                  upd_w2, upd_b2.reshape(1, C))
    return out
